# baseline (speedup 1.0000x reference)
import functools

import jax
import jax.numpy as jnp
from jax import lax
from jax.experimental import pallas as pl
from jax.experimental.pallas import tpu as pltpu
from jax.experimental.pallas import tpu_sc as plsc

_NUM_CORES = 2
_NUM_SUBCORES = 16
_NW = _NUM_CORES * _NUM_SUBCORES
_LANES = 16
_NBUF = 4


@functools.lru_cache(maxsize=None)
def _make_sc_partials(T):
    rows_per_w = T // _NW
    nchunk = T // _LANES
    mesh = plsc.VectorSubcoreMesh(core_axis_name="c", subcore_axis_name="s")

    @functools.partial(
        pl.kernel,
        out_type=jax.ShapeDtypeStruct((_NW, 2 * T), jnp.float32),
        mesh=mesh,
        scratch_types=[
            pltpu.VMEM((_NBUF * T,), jnp.float32),
            pltpu.VMEM((2 * T,), jnp.float32),
            [pltpu.SemaphoreType.DMA] * _NBUF,
        ],
        compiler_params=pltpu.CompilerParams(needs_layout_passes=False),
    )
    def sc_partials(x_hbm, part_hbm, buf, acc, sems):
        cid = lax.axis_index("c")
        sid = lax.axis_index("s")
        wid = cid * _NUM_SUBCORES + sid

        zero16 = jnp.zeros((_LANES,), jnp.float32)

        @plsc.parallel_loop(0, (2 * T) // _LANES, unroll=4)
        def _(k):
            acc[pl.ds(k * _LANES, _LANES)] = zero16

        def start_row_dma(m):
            i = wid + m * _NW
            return pltpu.async_copy(
                x_hbm.at[pl.ds(i * T, T)],
                buf.at[pl.ds((m % _NBUF) * T, T)],
                sems[m % _NBUF],
            )

        iota = lax.iota(jnp.int32, _LANES)
        copies = [None] * rows_per_w
        for m in range(_NBUF - 1):
            copies[m] = start_row_dma(m)

        for m in range(rows_per_w):
            if m + _NBUF - 1 < rows_per_w:
                copies[m + _NBUF - 1] = start_row_dma(m + _NBUF - 1)
            copies[m].wait()
            base = (m % _NBUF) * T

            i = wid + m * _NW
            shift = T - i
            lo = jnp.maximum(0, 2 * i - T + 2)
            c0 = lo // _LANES

            @pl.when(c0 < nchunk - 1)
            def _():
                jvec = iota + c0 * _LANES
                xv = buf[pl.ds(base + c0 * _LANES, _LANES)]
                plsc.addupdate_scatter(acc, [jvec + shift], xv, mask=jvec >= lo)

            @plsc.parallel_loop(c0 + 1, nchunk - 1, unroll=4)
            def _(k):
                xv = buf[pl.ds(base + k * _LANES, _LANES)]
                plsc.addupdate(acc.at[pl.ds(k * _LANES + shift, _LANES)], xv)

            jvec = iota + (nchunk - 1) * _LANES
            xv = buf[pl.ds(base + (nchunk - 1) * _LANES, _LANES)]
            plsc.addupdate_scatter(
                acc, [jvec + shift], xv, mask=(jvec >= lo) & (jvec < T - 1)
            )

        pltpu.sync_copy(acc, part_hbm.at[wid])

    return sc_partials


def _combine_body(T, p_ref, o_ref):
    s = jnp.sum(p_ref[...], axis=0, keepdims=True)
    b = lax.broadcasted_iota(jnp.int32, (1, 2 * T), 1)
    cnt = jnp.where(b < T, 2 * b - T - 1, 2 * T - 1 - b)
    cnt = jnp.maximum(cnt, 0).astype(jnp.float32)
    o_ref[...] = s / cnt


def kernel(inputs):
    T = inputs.shape[0]
    partials = _make_sc_partials(T)(inputs.reshape(T * T))
    out = pl.pallas_call(
        functools.partial(_combine_body, T),
        out_shape=jax.ShapeDtypeStruct((1, 2 * T), jnp.float32),
    )(partials)
    return out.reshape(2 * T)


# speedup vs baseline: 1.2034x; 1.2034x over previous
import functools

import jax
import jax.numpy as jnp
from jax import lax
from jax.experimental import pallas as pl
from jax.experimental.pallas import tpu as pltpu
from jax.experimental.pallas import tpu_sc as plsc

_NUM_CORES = 2
_NUM_SUBCORES = 16
_NW = _NUM_CORES * _NUM_SUBCORES
_LANES = 16
_BLK = 16


@functools.lru_cache(maxsize=None)
def _make_sc_partials(T):
    nchunk = T // _LANES
    nblk_total = T // _BLK
    mesh = plsc.VectorSubcoreMesh(core_axis_name="c", subcore_axis_name="s")

    @functools.partial(
        pl.kernel,
        out_type=jax.ShapeDtypeStruct((_NW, 2 * T), jnp.float32),
        mesh=mesh,
        scratch_types=[
            pltpu.VMEM((2 * _BLK * T,), jnp.float32),
            pltpu.VMEM((2 * T,), jnp.float32),
            [pltpu.SemaphoreType.DMA] * 2,
        ],
        compiler_params=pltpu.CompilerParams(needs_layout_passes=False),
    )
    def sc_partials(x_hbm, part_hbm, buf, acc, sems):
        cid = lax.axis_index("c")
        sid = lax.axis_index("s")
        wid = cid * _NUM_SUBCORES + sid

        zero16 = jnp.zeros((_LANES,), jnp.float32)

        @plsc.parallel_loop(0, (2 * T) // _LANES, unroll=4)
        def _(k):
            acc[pl.ds(k * _LANES, _LANES)] = zero16

        half = nblk_total // 2
        blocks = [
            wid,
            half - 1 - wid,
            half + wid,
            nblk_total - 1 - wid,
        ]

        iota = lax.iota(jnp.int32, _LANES)

        def start_blk_dma(idx, slot):
            blk_id = blocks[idx]
            return pltpu.async_copy(
                x_hbm.at[pl.ds(blk_id * _BLK * T, _BLK * T)],
                buf.at[pl.ds(slot * _BLK * T, _BLK * T)],
                sems[slot],
            )

        copies = [None] * len(blocks)
        copies[0] = start_blk_dma(0, 0)
        for bi in range(len(blocks)):
            if bi + 1 < len(blocks):
                copies[bi + 1] = start_blk_dma(bi + 1, (bi + 1) % 2)
            copies[bi].wait()
            slot_base = (bi % 2) * _BLK * T
            row0 = blocks[bi] * _BLK

            def row_body(r, carry):
                i = row0 + r
                base = slot_base + r * T
                shift = T - i
                lo = jnp.maximum(0, 2 * i - T + 2)
                c0 = lo // _LANES

                @pl.when(c0 < nchunk - 1)
                def _():
                    jvec = iota + c0 * _LANES
                    xv = buf[pl.ds(base + c0 * _LANES, _LANES)]
                    plsc.addupdate_scatter(
                        acc, [jvec + shift], xv, mask=jvec >= lo
                    )

                @plsc.parallel_loop(c0 + 1, nchunk - 1, unroll=4)
                def _(k):
                    xv = buf[pl.ds(base + k * _LANES, _LANES)]
                    plsc.addupdate(
                        acc.at[pl.ds(k * _LANES + shift, _LANES)], xv
                    )

                jvec = iota + (nchunk - 1) * _LANES
                xv = buf[pl.ds(base + (nchunk - 1) * _LANES, _LANES)]
                plsc.addupdate_scatter(
                    acc, [jvec + shift], xv, mask=(jvec >= lo) & (jvec < T - 1)
                )
                return carry

            lax.fori_loop(0, _BLK, row_body, 0)

        pltpu.sync_copy(acc, part_hbm.at[wid])

    return sc_partials


def _combine_body(T, p_ref, o_ref):
    s = jnp.sum(p_ref[...], axis=0, keepdims=True)
    b = lax.broadcasted_iota(jnp.int32, (1, 2 * T), 1)
    cnt = jnp.where(b < T, 2 * b - T - 1, 2 * T - 1 - b)
    cnt = jnp.maximum(cnt, 0).astype(jnp.float32)
    o_ref[...] = s / cnt


def kernel(inputs):
    T = inputs.shape[0]
    partials = _make_sc_partials(T)(inputs.reshape(T * T))
    out = pl.pallas_call(
        functools.partial(_combine_body, T),
        out_shape=jax.ShapeDtypeStruct((1, 2 * T), jnp.float32),
    )(partials)
    return out.reshape(2 * T)


# speedup vs baseline: 1.7200x; 1.4293x over previous
import functools

import jax
import jax.numpy as jnp
from jax import lax
from jax.experimental import pallas as pl
from jax.experimental.pallas import tpu as pltpu
from jax.experimental.pallas import tpu_sc as plsc

_NUM_CORES = 2
_NUM_SUBCORES = 16
_NW = _NUM_CORES * _NUM_SUBCORES
_LANES = 16
_BLK = 16


@functools.lru_cache(maxsize=None)
def _make_sc_partials(T):
    nchunk = T // _LANES
    nblk_total = T // _BLK
    mesh = plsc.VectorSubcoreMesh(core_axis_name="c", subcore_axis_name="s")

    @functools.partial(
        pl.kernel,
        out_type=jax.ShapeDtypeStruct((_NW, 2 * T), jnp.float32),
        mesh=mesh,
        scratch_types=[
            pltpu.VMEM((2 * _BLK, T), jnp.float32),
            pltpu.VMEM((2 * T,), jnp.float32),
            [pltpu.SemaphoreType.DMA] * 2,
        ],
        compiler_params=pltpu.CompilerParams(needs_layout_passes=False),
    )
    def sc_partials(x_hbm, part_hbm, buf, acc, sems):
        cid = lax.axis_index("c")
        sid = lax.axis_index("s")
        wid = cid * _NUM_SUBCORES + sid

        zero16 = jnp.zeros((_LANES,), jnp.float32)

        @plsc.parallel_loop(0, (2 * T) // _LANES, unroll=4)
        def _(k):
            acc[pl.ds(k * _LANES, _LANES)] = zero16

        half = nblk_total // 2
        blocks = [
            wid,
            half - 1 - wid,
            half + wid,
            nblk_total - 1 - wid,
        ]

        iota = lax.iota(jnp.int32, _LANES)

        def start_blk_dma(idx, slot):
            blk_id = blocks[idx]
            return pltpu.async_copy(
                x_hbm.at[pl.ds(blk_id * _BLK, _BLK), :],
                buf.at[pl.ds(slot * _BLK, _BLK), :],
                sems[slot],
            )

        copies = [None] * len(blocks)
        copies[0] = start_blk_dma(0, 0)
        for bi in range(len(blocks)):
            if bi + 1 < len(blocks):
                copies[bi + 1] = start_blk_dma(bi + 1, (bi + 1) % 2)
            copies[bi].wait()
            slot0 = (bi % 2) * _BLK
            row0 = blocks[bi] * _BLK

            def row_body(rr, carry):
                r = slot0 + rr
                i = row0 + rr
                shift = T - i
                lo = jnp.maximum(0, 2 * i - T + 2)
                c0 = lo // _LANES

                @pl.when(c0 < nchunk - 1)
                def _():
                    jvec = iota + c0 * _LANES
                    xv = buf[r, pl.ds(c0 * _LANES, _LANES)]
                    plsc.addupdate_scatter(
                        acc, [jvec + shift], xv, mask=jvec >= lo
                    )

                @plsc.parallel_loop(c0 + 1, nchunk - 1, unroll=4)
                def _(k):
                    xv = buf[r, pl.ds(k * _LANES, _LANES)]
                    plsc.addupdate(
                        acc.at[pl.ds(k * _LANES + shift, _LANES)], xv
                    )

                jvec = iota + (nchunk - 1) * _LANES
                xv = buf[r, pl.ds((nchunk - 1) * _LANES, _LANES)]
                plsc.addupdate_scatter(
                    acc, [jvec + shift], xv, mask=(jvec >= lo) & (jvec < T - 1)
                )
                return carry

            lax.fori_loop(0, _BLK, row_body, 0)

        pltpu.sync_copy(acc, part_hbm.at[wid])

    return sc_partials


def _combine_body(T, p_ref, o_ref):
    s = jnp.sum(p_ref[...], axis=0, keepdims=True)
    b = lax.broadcasted_iota(jnp.int32, (1, 2 * T), 1)
    cnt = jnp.where(b < T, 2 * b - T - 1, 2 * T - 1 - b)
    cnt = jnp.maximum(cnt, 0).astype(jnp.float32)
    o_ref[...] = s / cnt


def kernel(inputs):
    T = inputs.shape[0]
    partials = _make_sc_partials(T)(inputs)
    out = pl.pallas_call(
        functools.partial(_combine_body, T),
        out_shape=jax.ShapeDtypeStruct((1, 2 * T), jnp.float32),
    )(partials)
    return out.reshape(2 * T)
